# baseline (speedup 1.0000x reference)
import functools

import jax
import jax.numpy as jnp
from jax import lax
from jax.experimental import pallas as pl
from jax.experimental.pallas import tpu as pltpu
from jax.experimental.pallas import tpu_sc as plsc

_N = 50000
_E = 800000

_NC = 2
_NS = 16
_NW = _NC * _NS
_EPW = _E // _NW

_CHUNKS = {1: 1000, 16: 1000, 32: 200}

_R = 5000
_G = _N // _R


def _mesh():
    return plsc.VectorSubcoreMesh(
        core_axis_name="c", subcore_axis_name="s",
        num_cores=_NC, num_subcores=_NS)


def _row_chunks(s, nz, body):
    zc = _N // nz
    @pl.loop(0, -(-nz // _NS))
    def _(t):
        z = s + t * _NS
        @pl.when(z < nz)
        def _():
            body(pl.multiple_of(z * zc, 8))


def _make_agg(d):
    chunk = _CHUNKS[d]
    trips = _EPW // chunk
    pairs = -(-trips // 2)
    zc = min(chunk, 400)
    nz = _N // zc
    dd = (d,) if d > 1 else (1,)
    out_t = jax.ShapeDtypeStruct((_N,) + dd, jnp.float32)
    acc_t = pltpu.VMEM_SHARED((_N,) + dd, jnp.float32)
    rows_t = pltpu.VMEM((chunk,) + dd, jnp.float32)
    idx_t = pltpu.VMEM((chunk,), jnp.int32)

    @functools.partial(
        pl.kernel,
        out_type=(out_t, out_t),
        mesh=_mesh(),
        scratch_types=[
            acc_t, idx_t, idx_t, idx_t, idx_t, rows_t, rows_t,
            pltpu.SemaphoreType.DMA, pltpu.SemaphoreType.DMA,
            pltpu.SemaphoreType.DMA, pltpu.SemaphoreType.DMA,
            pltpu.SemaphoreType.DMA, pltpu.SemaphoreType.DMA,
        ],
        compiler_params=pltpu.CompilerParams(use_tc_tiling_on_sc=False),
    )
    def agg(u_hbm, zeros_hbm, src_hbm, dst_hbm, out0_hbm, out1_hbm,
            acc, isrc0, idst0, isrc1, idst1, rows0, rows1,
            si0, si1, sg0, sg1, ss0, ss1):
        c = lax.axis_index("c")
        s = lax.axis_index("s")
        w = s * _NC + c
        ebase = w * _EPW
        isrc = (isrc0, isrc1)
        idst = (idst0, idst1)
        rows = (rows0, rows1)
        si = (si0, si1)
        sg = (sg0, sg1)
        ss = (ss0, ss1)

        zview = rows0.at[pl.ds(0, zc)]
        pltpu.sync_copy(zeros_hbm, zview)
        _row_chunks(s, nz, lambda off: pltpu.sync_copy(
            zview, acc.at[pl.ds(off, zc)]))
        plsc.subcore_barrier()

        def idx_load(t, b):
            base = pl.multiple_of(ebase + t * chunk, 8)
            pltpu.async_copy(src_hbm.at[pl.ds(base, chunk)], isrc[b], si[b])
            pltpu.async_copy(dst_hbm.at[pl.ds(base, chunk)], idst[b], si[b])

        def idx_wait(b):
            pltpu.make_async_copy(
                src_hbm.at[pl.ds(0, chunk)], isrc[b], si[b]).wait()
            pltpu.make_async_copy(
                dst_hbm.at[pl.ds(0, chunk)], idst[b], si[b]).wait()

        def scat_wait(b):
            pltpu.make_async_copy(rows[b], acc.at[idst[b]], ss[b]).wait()

        idx_load(0, 0)

        @pl.loop(0, pairs)
        def _(p):
            for b in (0, 1):
                t = 2 * p + b
                @pl.when(t < trips)
                def _():
                    idx_wait(b)
                    pltpu.async_copy(u_hbm.at[isrc[b]], rows[b], sg[b])
                    pltpu.make_async_copy(
                        u_hbm.at[isrc[b]], rows[b], sg[b]).wait()
                    @pl.when(t >= 1)
                    def _():
                        scat_wait(1 - b)
                    pltpu.async_copy(rows[b], acc.at[idst[b]], ss[b], add=True)
                    @pl.when(t + 1 < trips)
                    def _():
                        idx_load(t + 1, 1 - b)

        scat_wait((trips - 1) % 2)
        plsc.subcore_barrier()

        def _out_to(out_hbm):
            def _out(off):
                pltpu.sync_copy(acc.at[pl.ds(off, zc)], zview)
                pltpu.sync_copy(zview, out_hbm.at[pl.ds(off, zc)])
            return _out

        @pl.when(c == 0)
        def _():
            _row_chunks(s, nz, _out_to(out0_hbm))

        @pl.when(c == 1)
        def _():
            _row_chunks(s, nz, _out_to(out1_hbm))

    return agg


def _make_deg():
    out_t = jax.ShapeDtypeStruct((_N, 1), jnp.float32)
    chunk = 1000
    trips = _EPW // chunk
    zc = 400
    nz = _N // zc

    @functools.partial(
        pl.kernel,
        out_type=(out_t, out_t, out_t, out_t),
        mesh=_mesh(),
        scratch_types=[
            pltpu.VMEM_SHARED((_N, 1), jnp.float32),
            pltpu.VMEM_SHARED((_N, 1), jnp.float32),
            pltpu.VMEM((chunk,), jnp.int32),
            pltpu.VMEM((chunk,), jnp.int32),
            pltpu.VMEM((chunk, 1), jnp.float32),
            pltpu.VMEM((zc, 1), jnp.float32),
        ],
        compiler_params=pltpu.CompilerParams(use_tc_tiling_on_sc=False),
    )
    def deg(ones_hbm, zeros_hbm, src_hbm, dst_hbm,
            od0_hbm, od1_hbm, id0_hbm, id1_hbm,
            acco, acci, isrc, idst, ones_v, zbuf):
        c = lax.axis_index("c")
        s = lax.axis_index("s")
        w = s * _NC + c
        ebase = w * _EPW

        pltpu.sync_copy(zeros_hbm, zbuf)
        pltpu.sync_copy(ones_hbm, ones_v)

        def _zero(off):
            pltpu.sync_copy(zbuf, acco.at[pl.ds(off, zc)])
            pltpu.sync_copy(zbuf, acci.at[pl.ds(off, zc)])
        _row_chunks(s, nz, _zero)
        plsc.subcore_barrier()

        @pl.loop(0, trips)
        def _(t):
            base = pl.multiple_of(ebase + t * chunk, 8)
            pltpu.sync_copy(src_hbm.at[pl.ds(base, chunk)], isrc)
            pltpu.sync_copy(dst_hbm.at[pl.ds(base, chunk)], idst)
            pltpu.sync_copy(ones_v, acco.at[isrc], add=True)
            pltpu.sync_copy(ones_v, acci.at[idst], add=True)

        plsc.subcore_barrier()

        def _out_to(od_hbm, id_hbm):
            def _out(off):
                pltpu.sync_copy(acco.at[pl.ds(off, zc)], zbuf)
                pltpu.sync_copy(zbuf, od_hbm.at[pl.ds(off, zc)])
                pltpu.sync_copy(acci.at[pl.ds(off, zc)], zbuf)
                pltpu.sync_copy(zbuf, id_hbm.at[pl.ds(off, zc)])
            return _out

        @pl.when(c == 0)
        def _():
            _row_chunks(s, nz, _out_to(od0_hbm, id0_hbm))

        @pl.when(c == 1)
        def _():
            _row_chunks(s, nz, _out_to(od1_hbm, id1_hbm))

    return deg


def _vspec(d):
    return pl.BlockSpec((_R, d), lambda i: (i, 0))


def _wspec(shape):
    ndim = len(shape)
    return pl.BlockSpec(shape, lambda i, _nd=ndim: (0,) * _nd)


def _f32(*shape):
    return jax.ShapeDtypeStruct(shape, jnp.float32)


def _tc_call(body, in_specs, out_specs, out_shape, args):
    if len(out_specs) == 1:
        out_specs = out_specs[0]
    return pl.pallas_call(
        body,
        grid=(_G,),
        in_specs=in_specs,
        out_specs=out_specs,
        out_shape=out_shape,
    )(*args)


def kernel(x, edge_index, W1, b1, W2, b2, W3, b3, W4, b4, W5, b5):
    src = edge_index[0]
    dst = edge_index[1]
    x2 = x.reshape(_N, 1)
    b1r, b2r, b3r, b4r, b5r = (b.reshape(1, -1) for b in (b1, b2, b3, b4, b5))
    ones_in = jnp.ones((1000, 1), jnp.float32)
    z400 = jnp.zeros((400, 1), jnp.float32)
    z400_16 = jnp.zeros((400, 16), jnp.float32)
    z200_32 = jnp.zeros((200, 32), jnp.float32)

    agg1 = _make_agg(1)
    agg16 = _make_agg(16)
    agg32 = _make_agg(32)

    od0, od1, id0, id1 = _make_deg()(ones_in, z400, src, dst)

    def t0(od0_r, od1_r, id0_r, id1_r, x_r, on_o, in_o, u1_o):
        od = od0_r[...] + od1_r[...]
        ideg = id0_r[...] + id1_r[...]
        on_o[...] = jnp.where(od > 0, lax.rsqrt(jnp.maximum(od, 1.0)), 0.0)
        in_o[...] = jnp.where(ideg > 0, lax.rsqrt(jnp.maximum(ideg, 1.0)), 0.0)
        u1_o[...] = on_o[...] * x_r[...]

    on, inn, u1 = _tc_call(
        t0,
        [_vspec(1)] * 5,
        [_vspec(1)] * 3,
        (_f32(_N, 1), _f32(_N, 1), _f32(_N, 1)),
        (od0, od1, id0, id1, x2))

    s10, s11 = agg1(u1, z400, src, dst)

    def t1(s0_r, s1_r, in_r, on_r, w_r, b_r, u_o):
        sv = in_r[...] * (s0_r[...] + s1_r[...])
        u_o[...] = on_r[...] * jax.nn.relu(sv * w_r[...] + b_r[...])

    u2 = _tc_call(
        t1,
        [_vspec(1)] * 4 + [_wspec((1, 16)), _wspec((1, 16))],
        [_vspec(16)],
        _f32(_N, 16),
        (s10, s11, inn, on, W1, b1r))

    s20, s21 = agg16(u2, z400_16, src, dst)

    def t2(s0_r, s1_r, in_r, on_r, w_r, b_r, u_o):
        sv = in_r[...] * (s0_r[...] + s1_r[...])
        h = jax.nn.relu(jnp.dot(sv, w_r[...],
                                preferred_element_type=jnp.float32) + b_r[...])
        u_o[...] = on_r[...] * h

    u3 = _tc_call(
        t2,
        [_vspec(16), _vspec(16), _vspec(1), _vspec(1),
         _wspec((16, 32)), _wspec((1, 32))],
        [_vspec(32)],
        _f32(_N, 32),
        (s20, s21, inn, on, W2, b2r))

    s30, s31 = agg32(u3, z200_32, src, dst)

    def t3(s0_r, s1_r, in_r, on_r, w3_r, b3_r, w4_r, u_o):
        sv = in_r[...] * (s0_r[...] + s1_r[...])
        h = jax.nn.relu(jnp.dot(sv, w3_r[...],
                                preferred_element_type=jnp.float32) + b3_r[...])
        u_o[...] = on_r[...] * jnp.dot(h, w4_r[...],
                                       preferred_element_type=jnp.float32)

    u4 = _tc_call(
        t3,
        [_vspec(32), _vspec(32), _vspec(1), _vspec(1),
         _wspec((32, 64)), _wspec((1, 64)), _wspec((64, 32))],
        [_vspec(32)],
        _f32(_N, 32),
        (s30, s31, inn, on, W3, b3r, W4))

    s40, s41 = agg32(u4, z200_32, src, dst)

    def t4(s0_r, s1_r, in_r, on_r, b4_r, w5_r, u_o):
        h = jax.nn.relu(in_r[...] * (s0_r[...] + s1_r[...]) + b4_r[...])
        u_o[...] = on_r[...] * jnp.dot(h, w5_r[...],
                                       preferred_element_type=jnp.float32)

    u5 = _tc_call(
        t4,
        [_vspec(32), _vspec(32), _vspec(1), _vspec(1),
         _wspec((1, 32)), _wspec((32, 1))],
        [_vspec(1)],
        _f32(_N, 1),
        (s40, s41, inn, on, b4r, W5))

    s50, s51 = agg1(u5, z400, src, dst)

    def t5(s0_r, s1_r, in_r, b_r, y_o):
        y_o[...] = jax.nn.sigmoid(
            in_r[...] * (s0_r[...] + s1_r[...]) + b_r[...])

    y = _tc_call(
        t5,
        [_vspec(1)] * 3 + [_wspec((1, 1))],
        [_vspec(1)],
        _f32(_N, 1),
        (s50, s51, inn, b5r))

    return y.reshape(1, _N)


# speedup vs baseline: 1.3130x; 1.3130x over previous
import functools

import jax
import jax.numpy as jnp
from jax import lax
from jax.experimental import pallas as pl
from jax.experimental.pallas import tpu as pltpu
from jax.experimental.pallas import tpu_sc as plsc

_N = 50000
_E = 800000

_NC = 2
_NS = 16
_NW = _NC * _NS
_EPW = _E // _NW

_CHUNKS = {1: 1000, 16: 1000, 32: 200}

_R = 5000
_G = _N // _R


def _mesh():
    return plsc.VectorSubcoreMesh(
        core_axis_name="c", subcore_axis_name="s",
        num_cores=_NC, num_subcores=_NS)


def _row_chunks(s, nz, body):
    zc = _N // nz
    @pl.loop(0, -(-nz // _NS))
    def _(t):
        z = s + t * _NS
        @pl.when(z < nz)
        def _():
            body(pl.multiple_of(z * zc, 8))


def _make_agg(d):
    chunk = _CHUNKS[d]
    trips = _EPW // chunk
    pairs = -(-trips // 2)
    zc = min(chunk, 400)
    nz = _N // zc
    dd = (d,) if d > 1 else ()
    out_t = jax.ShapeDtypeStruct((_N,) + dd, jnp.float32)
    acc_t = pltpu.VMEM_SHARED((_N,) + dd, jnp.float32)
    rows_t = pltpu.VMEM((chunk,) + dd, jnp.float32)
    idx_t = pltpu.VMEM((chunk,), jnp.int32)

    @functools.partial(
        pl.kernel,
        out_type=(out_t, out_t),
        mesh=_mesh(),
        scratch_types=[
            acc_t, idx_t, idx_t, idx_t, idx_t, rows_t, rows_t,
            pltpu.SemaphoreType.DMA, pltpu.SemaphoreType.DMA,
            pltpu.SemaphoreType.DMA, pltpu.SemaphoreType.DMA,
            pltpu.SemaphoreType.DMA, pltpu.SemaphoreType.DMA,
        ],
        compiler_params=pltpu.CompilerParams(use_tc_tiling_on_sc=False),
    )
    def agg(u_hbm, zeros_hbm, src_hbm, dst_hbm, out0_hbm, out1_hbm,
            acc, isrc0, idst0, isrc1, idst1, rows0, rows1,
            si0, si1, sg0, sg1, ss0, ss1):
        c = lax.axis_index("c")
        s = lax.axis_index("s")
        w = s * _NC + c
        ebase = w * _EPW
        isrc = (isrc0, isrc1)
        idst = (idst0, idst1)
        rows = (rows0, rows1)
        si = (si0, si1)
        sg = (sg0, sg1)
        ss = (ss0, ss1)

        zview = rows0.at[pl.ds(0, zc)]
        pltpu.sync_copy(zeros_hbm, zview)
        _row_chunks(s, nz, lambda off: pltpu.sync_copy(
            zview, acc.at[pl.ds(off, zc)]))
        plsc.subcore_barrier()

        def idx_load(t, b):
            base = pl.multiple_of(ebase + t * chunk, 8)
            pltpu.async_copy(src_hbm.at[pl.ds(base, chunk)], isrc[b], si[b])
            pltpu.async_copy(dst_hbm.at[pl.ds(base, chunk)], idst[b], si[b])

        def idx_wait(b):
            pltpu.make_async_copy(
                src_hbm.at[pl.ds(0, chunk)], isrc[b], si[b]).wait()
            pltpu.make_async_copy(
                dst_hbm.at[pl.ds(0, chunk)], idst[b], si[b]).wait()

        def scat_wait(b):
            pltpu.make_async_copy(rows[b], acc.at[idst[b]], ss[b]).wait()

        idx_load(0, 0)

        @pl.loop(0, pairs)
        def _(p):
            for b in (0, 1):
                t = 2 * p + b
                @pl.when(t < trips)
                def _():
                    idx_wait(b)
                    pltpu.async_copy(u_hbm.at[isrc[b]], rows[b], sg[b])
                    pltpu.make_async_copy(
                        u_hbm.at[isrc[b]], rows[b], sg[b]).wait()
                    @pl.when(t >= 1)
                    def _():
                        scat_wait(1 - b)
                    pltpu.async_copy(rows[b], acc.at[idst[b]], ss[b], add=True)
                    @pl.when(t + 1 < trips)
                    def _():
                        idx_load(t + 1, 1 - b)

        scat_wait((trips - 1) % 2)
        plsc.subcore_barrier()

        def _out_to(out_hbm):
            def _out(off):
                pltpu.sync_copy(acc.at[pl.ds(off, zc)], zview)
                pltpu.sync_copy(zview, out_hbm.at[pl.ds(off, zc)])
            return _out

        @pl.when(c == 0)
        def _():
            _row_chunks(s, nz, _out_to(out0_hbm))

        @pl.when(c == 1)
        def _():
            _row_chunks(s, nz, _out_to(out1_hbm))

    return agg


def _make_deg():
    out_t = jax.ShapeDtypeStruct((_N,), jnp.float32)
    chunk = 1000
    trips = _EPW // chunk
    zc = 400
    nz = _N // zc

    @functools.partial(
        pl.kernel,
        out_type=(out_t, out_t, out_t, out_t),
        mesh=_mesh(),
        scratch_types=[
            pltpu.VMEM_SHARED((_N,), jnp.float32),
            pltpu.VMEM_SHARED((_N,), jnp.float32),
            pltpu.VMEM((chunk,), jnp.int32),
            pltpu.VMEM((chunk,), jnp.int32),
            pltpu.VMEM((chunk,), jnp.float32),
            pltpu.VMEM((zc,), jnp.float32),
        ],
        compiler_params=pltpu.CompilerParams(use_tc_tiling_on_sc=False),
    )
    def deg(ones_hbm, zeros_hbm, src_hbm, dst_hbm,
            od0_hbm, od1_hbm, id0_hbm, id1_hbm,
            acco, acci, isrc, idst, ones_v, zbuf):
        c = lax.axis_index("c")
        s = lax.axis_index("s")
        w = s * _NC + c
        ebase = w * _EPW

        pltpu.sync_copy(zeros_hbm, zbuf)
        pltpu.sync_copy(ones_hbm, ones_v)

        def _zero(off):
            pltpu.sync_copy(zbuf, acco.at[pl.ds(off, zc)])
            pltpu.sync_copy(zbuf, acci.at[pl.ds(off, zc)])
        _row_chunks(s, nz, _zero)
        plsc.subcore_barrier()

        @pl.loop(0, trips)
        def _(t):
            base = pl.multiple_of(ebase + t * chunk, 8)
            pltpu.sync_copy(src_hbm.at[pl.ds(base, chunk)], isrc)
            pltpu.sync_copy(dst_hbm.at[pl.ds(base, chunk)], idst)
            pltpu.sync_copy(ones_v, acco.at[isrc], add=True)
            pltpu.sync_copy(ones_v, acci.at[idst], add=True)

        plsc.subcore_barrier()

        def _out_to(od_hbm, id_hbm):
            def _out(off):
                pltpu.sync_copy(acco.at[pl.ds(off, zc)], zbuf)
                pltpu.sync_copy(zbuf, od_hbm.at[pl.ds(off, zc)])
                pltpu.sync_copy(acci.at[pl.ds(off, zc)], zbuf)
                pltpu.sync_copy(zbuf, id_hbm.at[pl.ds(off, zc)])
            return _out

        @pl.when(c == 0)
        def _():
            _row_chunks(s, nz, _out_to(od0_hbm, id0_hbm))

        @pl.when(c == 1)
        def _():
            _row_chunks(s, nz, _out_to(od1_hbm, id1_hbm))

    return deg


def _vspec(d):
    return pl.BlockSpec((_R, d), lambda i: (i, 0))


def _wspec(shape):
    ndim = len(shape)
    return pl.BlockSpec(shape, lambda i, _nd=ndim: (0,) * _nd)


def _f32(*shape):
    return jax.ShapeDtypeStruct(shape, jnp.float32)


def _tc_call(body, in_specs, out_specs, out_shape, args):
    if len(out_specs) == 1:
        out_specs = out_specs[0]
    return pl.pallas_call(
        body,
        grid=(_G,),
        in_specs=in_specs,
        out_specs=out_specs,
        out_shape=out_shape,
    )(*args)


def _tc1d(body, out_shape, args):
    return pl.pallas_call(body, out_shape=out_shape)(*args)


def kernel(x, edge_index, W1, b1, W2, b2, W3, b3, W4, b4, W5, b5):
    src = edge_index[0]
    dst = edge_index[1]
    x1 = x.reshape(_N)
    b1r, b2r, b3r, b4r, b5r = (b.reshape(1, -1) for b in (b1, b2, b3, b4, b5))
    ones_in = jnp.ones((1000,), jnp.float32)
    z400 = jnp.zeros((400,), jnp.float32)
    z400_16 = jnp.zeros((400, 16), jnp.float32)
    z200_32 = jnp.zeros((200, 32), jnp.float32)

    agg1 = _make_agg(1)
    agg16 = _make_agg(16)
    agg32 = _make_agg(32)

    od0, od1, id0, id1 = _make_deg()(ones_in, z400, src, dst)

    def t0(od0_r, od1_r, id0_r, id1_r, x_r, on_o, in_o, u1_o):
        od = od0_r[...] + od1_r[...]
        ideg = id0_r[...] + id1_r[...]
        on_o[...] = jnp.where(od > 0, lax.rsqrt(jnp.maximum(od, 1.0)), 0.0)
        in_o[...] = jnp.where(ideg > 0, lax.rsqrt(jnp.maximum(ideg, 1.0)), 0.0)
        u1_o[...] = on_o[...] * x_r[...]

    on1, in1, u1 = _tc1d(
        t0, (_f32(_N), _f32(_N), _f32(_N)), (od0, od1, id0, id1, x1))

    s10, s11 = agg1(u1, z400, src, dst)

    def tsv(s0_r, s1_r, in_r, sv_o):
        sv_o[...] = in_r[...] * (s0_r[...] + s1_r[...])

    sv1 = _tc1d(tsv, _f32(_N), (s10, s11, in1))
    sv1c = sv1.reshape(_N, 1)
    onc = on1.reshape(_N, 1)
    innc = in1.reshape(_N, 1)

    def t1(sv_r, on_r, w_r, b_r, u_o):
        u_o[...] = on_r[...] * jax.nn.relu(sv_r[...] * w_r[...] + b_r[...])

    u2 = _tc_call(
        t1,
        [_vspec(1), _vspec(1), _wspec((1, 16)), _wspec((1, 16))],
        [_vspec(16)],
        _f32(_N, 16),
        (sv1c, onc, W1, b1r))

    s20, s21 = agg16(u2, z400_16, src, dst)

    def t2(s0_r, s1_r, in_r, on_r, w_r, b_r, u_o):
        sv = in_r[...] * (s0_r[...] + s1_r[...])
        h = jax.nn.relu(jnp.dot(sv, w_r[...],
                                preferred_element_type=jnp.float32) + b_r[...])
        u_o[...] = on_r[...] * h

    u3 = _tc_call(
        t2,
        [_vspec(16), _vspec(16), _vspec(1), _vspec(1),
         _wspec((16, 32)), _wspec((1, 32))],
        [_vspec(32)],
        _f32(_N, 32),
        (s20, s21, innc, onc, W2, b2r))

    s30, s31 = agg32(u3, z200_32, src, dst)

    def t3(s0_r, s1_r, in_r, on_r, w3_r, b3_r, w4_r, u_o):
        sv = in_r[...] * (s0_r[...] + s1_r[...])
        h = jax.nn.relu(jnp.dot(sv, w3_r[...],
                                preferred_element_type=jnp.float32) + b3_r[...])
        u_o[...] = on_r[...] * jnp.dot(h, w4_r[...],
                                       preferred_element_type=jnp.float32)

    u4 = _tc_call(
        t3,
        [_vspec(32), _vspec(32), _vspec(1), _vspec(1),
         _wspec((32, 64)), _wspec((1, 64)), _wspec((64, 32))],
        [_vspec(32)],
        _f32(_N, 32),
        (s30, s31, innc, onc, W3, b3r, W4))

    s40, s41 = agg32(u4, z200_32, src, dst)

    def t4(s0_r, s1_r, in_r, on_r, b4_r, w5_r, u_o):
        h = jax.nn.relu(in_r[...] * (s0_r[...] + s1_r[...]) + b4_r[...])
        u_o[...] = on_r[...] * jnp.dot(h, w5_r[...],
                                       preferred_element_type=jnp.float32)

    u5 = _tc_call(
        t4,
        [_vspec(32), _vspec(32), _vspec(1), _vspec(1),
         _wspec((1, 32)), _wspec((32, 1))],
        [_vspec(1)],
        _f32(_N, 1),
        (s40, s41, innc, onc, b4r, W5))

    s50, s51 = agg1(u5.reshape(_N), z400, src, dst)

    def t5(s0_r, s1_r, in_r, b_r, y_o):
        y_o[...] = jax.nn.sigmoid(
            in_r[...] * (s0_r[...] + s1_r[...]) + b_r[0])

    y = _tc1d(t5, _f32(_N), (s50, s51, in1, b5))

    return y.reshape(1, _N)
